# baseline (speedup 1.0000x reference)
import jax
import jax.numpy as jnp
import numpy as np
from jax.experimental import pallas as pl
from jax.experimental.pallas import tpu as pltpu

_M = 128
_BN = 3584
_NSLAB = _M // 8

_PERM = np.array([(p % 8) * 16 + p // 8 for p in range(_M)], dtype=np.int32)


def _body(rad_ref, pts_ref, rois_ref, sampled_ref, mind_ref, mask_ref):
    pts = pts_ref[:, :]
    ptst = jnp.transpose(pts)
    px = ptst[0:1, :]
    py = ptst[1:2, :]
    pz = ptst[2:3, :]

    hx = rois_ref[:, 3:4] * jnp.float32(0.5)
    hy = rois_ref[:, 4:5] * jnp.float32(0.5)
    hz = rois_ref[:, 5:6] * jnp.float32(0.5)
    rnorm = jnp.sqrt((hx * hx + hy * hy) + hz * hz)

    nchain = 4
    per = _NSLAB // nchain
    ms = [None] * nchain
    tvs = [None] * nchain
    for c in range(nchain):
        for k in range(per):
            i = c * per + k
            cx = rois_ref[8 * i:8 * i + 8, 0:1]
            cy = rois_ref[8 * i:8 * i + 8, 1:2]
            cz = rois_ref[8 * i:8 * i + 8, 2:3]
            dx = px - cx
            dy = py - cy
            dz = pz - cz
            d2 = (dx * dx + dy * dy) + dz * dz
            rn = rnorm[8 * i:8 * i + 8, 0:1]
            if k == 0:
                ms[c] = d2
                tvs[c] = jnp.broadcast_to(rn, d2.shape)
            else:
                lt = d2 < ms[c]
                ms[c] = jnp.where(lt, d2, ms[c])
                tvs[c] = jnp.where(lt, rn, tvs[c])
    while len(ms) > 1:
        nms, ntvs = [], []
        for c in range(0, len(ms), 2):
            lt = ms[c + 1] < ms[c]
            nms.append(jnp.where(lt, ms[c + 1], ms[c]))
            ntvs.append(jnp.where(lt, tvs[c + 1], tvs[c]))
        ms, tvs = nms, ntvs
    m, tv = ms[0], tvs[0]

    for h in (4, 2, 1):
        m_lo, m_hi = m[:h, :], m[h:2 * h, :]
        tv_lo, tv_hi = tv[:h, :], tv[h:2 * h, :]
        lt = m_hi < m_lo
        m = jnp.where(lt, m_hi, m_lo)
        tv = jnp.where(lt, tv_hi, tv_lo)

    min_dis = jnp.sqrt(m + jnp.float32(1e-12))
    mask = min_dis < tv + rad_ref[0]

    mind_ref[:, :] = min_dis
    mask_ref[:, :] = mask
    maskf = jnp.where(mask, jnp.float32(1.0), jnp.float32(0.0))
    sampled_ref[:, :] = pts * jnp.transpose(maskf)


@jax.jit
def _run(points, rois, rad):
    n = points.shape[0]
    rois_k = rois[_PERM]
    grid = pl.cdiv(n, _BN)

    sampled, mind, mask = pl.pallas_call(
        _body,
        grid=(grid,),
        in_specs=[
            pl.BlockSpec(memory_space=pltpu.SMEM),
            pl.BlockSpec((_BN, 3), lambda i: (i, 0)),
            pl.BlockSpec((_M, 7), lambda i: (0, 0)),
        ],
        out_specs=[
            pl.BlockSpec((_BN, 3), lambda i: (i, 0)),
            pl.BlockSpec((1, _BN), lambda i: (0, i)),
            pl.BlockSpec((1, _BN), lambda i: (0, i)),
        ],
        out_shape=[
            jax.ShapeDtypeStruct((n, 3), jnp.float32),
            jax.ShapeDtypeStruct((1, n), jnp.float32),
            jax.ShapeDtypeStruct((1, n), jnp.bool_),
        ],
    )(rad, points, rois_k)
    return (sampled, mind[0], mask[0])


def kernel(points, rois, sample_radius_with_roi):
    rad = jnp.float32(sample_radius_with_roi).reshape((1,))
    return _run(points, rois, rad)


# speedup vs baseline: 2.2086x; 2.2086x over previous
import jax
import jax.numpy as jnp
import numpy as np
from jax.experimental import pallas as pl
from jax.experimental.pallas import tpu as pltpu

_M = 128
_BN = 3584
_NPAD = 100352
_NSLAB = _M // 8

_PERM = np.array([(p % 8) * 16 + p // 8 for p in range(_M)], dtype=np.int32)


def _body(rad_ref, pts_ref, rois_ref, sampled_ref, mind_ref, mask_ref):
    px = pts_ref[0:1, :]
    py = pts_ref[1:2, :]
    pz = pts_ref[2:3, :]

    hx = rois_ref[:, 3:4] * jnp.float32(0.5)
    hy = rois_ref[:, 4:5] * jnp.float32(0.5)
    hz = rois_ref[:, 5:6] * jnp.float32(0.5)
    rnorm = jnp.sqrt((hx * hx + hy * hy) + hz * hz)

    nchain = 4
    per = _NSLAB // nchain
    ms = [None] * nchain
    tvs = [None] * nchain
    for c in range(nchain):
        for k in range(per):
            i = c * per + k
            cx = rois_ref[8 * i:8 * i + 8, 0:1]
            cy = rois_ref[8 * i:8 * i + 8, 1:2]
            cz = rois_ref[8 * i:8 * i + 8, 2:3]
            dx = px - cx
            dy = py - cy
            dz = pz - cz
            d2 = (dx * dx + dy * dy) + dz * dz
            rn = rnorm[8 * i:8 * i + 8, 0:1]
            if k == 0:
                ms[c] = d2
                tvs[c] = jnp.broadcast_to(rn, d2.shape)
            else:
                lt = d2 < ms[c]
                ms[c] = jnp.where(lt, d2, ms[c])
                tvs[c] = jnp.where(lt, rn, tvs[c])
    while len(ms) > 1:
        nms, ntvs = [], []
        for c in range(0, len(ms), 2):
            lt = ms[c + 1] < ms[c]
            nms.append(jnp.where(lt, ms[c + 1], ms[c]))
            ntvs.append(jnp.where(lt, tvs[c + 1], tvs[c]))
        ms, tvs = nms, ntvs
    m, tv = ms[0], tvs[0]

    for h in (4, 2, 1):
        m_lo, m_hi = m[:h, :], m[h:2 * h, :]
        tv_lo, tv_hi = tv[:h, :], tv[h:2 * h, :]
        lt = m_hi < m_lo
        m = jnp.where(lt, m_hi, m_lo)
        tv = jnp.where(lt, tv_hi, tv_lo)

    min_dis = jnp.sqrt(m + jnp.float32(1e-12))
    mask = min_dis < tv + rad_ref[0]

    mind_ref[:, :] = min_dis
    mask_ref[:, :] = mask
    sampled_ref[:, :] = jnp.where(mask, pts_ref[:, :], jnp.float32(0.0))


@jax.jit
def _run(points, rois, rad):
    n = points.shape[0]
    pts_t = jnp.pad(points.T, ((0, 0), (0, _NPAD - n)))
    rois_k = rois[_PERM]
    grid = _NPAD // _BN

    sampled_t, mind, mask = pl.pallas_call(
        _body,
        grid=(grid,),
        in_specs=[
            pl.BlockSpec(memory_space=pltpu.SMEM),
            pl.BlockSpec((3, _BN), lambda i: (0, i)),
            pl.BlockSpec((_M, 7), lambda i: (0, 0)),
        ],
        out_specs=[
            pl.BlockSpec((3, _BN), lambda i: (0, i)),
            pl.BlockSpec((1, _BN), lambda i: (0, i)),
            pl.BlockSpec((1, _BN), lambda i: (0, i)),
        ],
        out_shape=[
            jax.ShapeDtypeStruct((3, _NPAD), jnp.float32),
            jax.ShapeDtypeStruct((1, _NPAD), jnp.float32),
            jax.ShapeDtypeStruct((1, _NPAD), jnp.bool_),
        ],
    )(rad, pts_t, rois_k)
    return (sampled_t[:, :n].T, mind[0, :n], mask[0, :n])


def kernel(points, rois, sample_radius_with_roi):
    rad = jnp.float32(sample_radius_with_roi).reshape((1,))
    return _run(points, rois, rad)
